# baseline (speedup 1.0000x reference)
import functools

import jax
import jax.numpy as jnp
from jax import lax
from jax.experimental import pallas as pl
from jax.experimental.pallas import tpu as pltpu
import jax.experimental.pallas.tpu_sc as plsc

B = 16384
F = 26
V = 100000
D = 16

NC = 2
NS = 16
NW = NC * NS
L = 16

FD = F * D
RPW = FD // NW
WCH = 2048
NCH = B // WCH
NWIDE = F * NCH
KMAX = RPW + (NWIDE + NW - 1) // NW
SUB = 128


def _sc_wide(idx_t, w2):
  mesh = plsc.VectorSubcoreMesh(
      core_axis_name="c", subcore_axis_name="s", num_cores=NC, num_subcores=NS
  )

  @functools.partial(
      pl.kernel,
      out_type=jax.ShapeDtypeStruct((F, B), jnp.float32),
      mesh=mesh,
      compiler_params=pltpu.CompilerParams(use_tc_tiling_on_sc=False),
      scratch_types=[
          pltpu.VMEM((WCH,), jnp.int32),
          pltpu.VMEM((WCH,), jnp.float32),
          pltpu.SemaphoreType.DMA,
      ],
  )
  def k(idxt, w, wT, idx_v, out_v, sem):
    wid = lax.axis_index("s") * NC + lax.axis_index("c")

    def wide_task(wt):
      f = wt // NCH
      c = wt % NCH
      pltpu.sync_copy(idxt.at[f, pl.ds(c * WCH, WCH)], idx_v)

      def addoff(i, carry):
        idx_v[pl.ds(i * L, L)] = idx_v[pl.ds(i * L, L)] + f * V
        return carry

      lax.fori_loop(0, WCH // L, addoff, 0)

      def subgather(g, carry):
        base = g * (8 * SUB)
        cps = []
        for t in range(8):
          ii = idx_v.at[pl.ds(base + t * SUB, SUB)]
          cps.append(pltpu.async_copy(
              w.at[0].at[ii], out_v.at[pl.ds(base + t * SUB, SUB)], sem))
        for cp in cps:
          cp.wait()
        return carry

      lax.fori_loop(0, WCH // (8 * SUB), subgather, 0)
      pltpu.sync_copy(out_v, wT.at[f, pl.ds(c * WCH, WCH)])

    def round_body(kk, carry):
      wt = wid + NW * kk

      @pl.when(wt < NWIDE)
      def _():
        wide_task(wt)

      return carry

    lax.fori_loop(0, (NWIDE + NW - 1) // NW, round_body, 0)

  return k(idx_t, w2)


def _sc_deep(table2, idx_t, fd_base, rpw):
  nrows = rpw * NW
  mesh = plsc.VectorSubcoreMesh(
      core_axis_name="c", subcore_axis_name="s", num_cores=NC, num_subcores=NS
  )

  @functools.partial(
      pl.kernel,
      out_type=jax.ShapeDtypeStruct((nrows, B), jnp.float32),
      mesh=mesh,
      compiler_params=pltpu.CompilerParams(use_tc_tiling_on_sc=False),
      scratch_types=[
          pltpu.VMEM_SHARED((NS, V), jnp.float32),
          pltpu.VMEM((2, WCH), jnp.int32),
          pltpu.VMEM((2, WCH), jnp.float32),
          pltpu.SemaphoreType.DMA,
          pltpu.SemaphoreType.DMA,
          pltpu.SemaphoreType.DMA,
      ],
  )
  def k(t2, idxt, xT, tab_s, idx_v, out_v, sem_g, sem_i, sem_o):
    sid = lax.axis_index("s")
    wid = sid * NC + lax.axis_index("c")
    my_tab = tab_s.at[sid]

    def deep_row(r, carry):
      fd = wid * rpw + r
      f = (fd_base + fd) // D
      pltpu.sync_copy(t2.at[fd], my_tab)
      pltpu.async_copy(idxt.at[f, pl.ds(0, WCH)], idx_v.at[0], sem_i).wait()

      def chunk(c, carry):
        cur = lax.rem(c, 2)
        nxt = lax.rem(c + 1, 2)

        @pl.when(c + 1 < NCH)
        def _():
          pltpu.async_copy(
              idxt.at[f, pl.ds((c + 1) * WCH, WCH)], idx_v.at[nxt], sem_i)

        def subgather(g, carry2):
          base = g * (8 * SUB)
          cps = []
          for t in range(8):
            ii = idx_v.at[cur].at[pl.ds(base + t * SUB, SUB)]
            cps.append(pltpu.async_copy(
                my_tab.at[ii], out_v.at[cur].at[pl.ds(base + t * SUB, SUB)],
                sem_g))
          for cp in cps:
            cp.wait()
          return carry2

        lax.fori_loop(0, WCH // (8 * SUB), subgather, 0)
        pltpu.sync_copy(out_v.at[cur], xT.at[fd, pl.ds(c * WCH, WCH)])

        @pl.when(c + 1 < NCH)
        def _():
          pltpu.make_async_copy(
              idxt.at[f, pl.ds((c + 1) * WCH, WCH)], idx_v.at[nxt], sem_i
          ).wait()

        return carry

      lax.fori_loop(0, NCH, chunk, 0)
      return carry

    lax.fori_loop(0, rpw, deep_row, 0)

  return k(table2, idx_t)


BM = 1024


NRA = 224
NRB = FD - NRA


def _mlp_body(xta_ref, xtb_ref, wvt_ref, W1_ref, b1_ref, W2_ref, b2_ref,
              W3_ref, b3_ref, Wf_ref, bf_ref, out_ref):
  cdims = (((0,), (0,)), ((), ()))
  W1 = W1_ref[...]
  pre = (lax.dot_general(xta_ref[...], W1[0:NRA, :], cdims,
                         preferred_element_type=jnp.float32)
         + lax.dot_general(xtb_ref[...], W1[NRA:FD, :], cdims,
                           preferred_element_type=jnp.float32))
  h = jnp.maximum(pre + b1_ref[...], 0.0)
  h = jnp.maximum(jnp.dot(h, W2_ref[...], preferred_element_type=jnp.float32)
                  + b2_ref[...], 0.0)
  h = jnp.maximum(jnp.dot(h, W3_ref[...], preferred_element_type=jnp.float32)
                  + b3_ref[...], 0.0)
  deep = jnp.dot(h, Wf_ref[...], preferred_element_type=jnp.float32) + bf_ref[...]
  ones = jnp.full((F, 1), 1.0, dtype=jnp.float32)
  wide = lax.dot_general(wvt_ref[...], ones, cdims,
                         preferred_element_type=jnp.float32)
  z = 0.5 * wide + 0.5 * deep
  out_ref[...] = 1.0 / (1.0 + jnp.exp(-z))


def _tc_mlp(xta, xtb, wvt, W1, b1, W2, b2, W3, b3, Wf, bf):
  grid = (B // BM,)
  return pl.pallas_call(
      _mlp_body,
      grid=grid,
      in_specs=[
          pl.BlockSpec((NRA, BM), lambda i: (0, i)),
          pl.BlockSpec((NRB, BM), lambda i: (0, i)),
          pl.BlockSpec((F, BM), lambda i: (0, i)),
          pl.BlockSpec((FD, 256), lambda i: (0, 0)),
          pl.BlockSpec((1, 256), lambda i: (0, 0)),
          pl.BlockSpec((256, 128), lambda i: (0, 0)),
          pl.BlockSpec((1, 128), lambda i: (0, 0)),
          pl.BlockSpec((128, 64), lambda i: (0, 0)),
          pl.BlockSpec((1, 64), lambda i: (0, 0)),
          pl.BlockSpec((64, 1), lambda i: (0, 0)),
          pl.BlockSpec((1, 1), lambda i: (0, 0)),
      ],
      out_specs=pl.BlockSpec((BM, 1), lambda i: (i, 0)),
      out_shape=jax.ShapeDtypeStruct((B, 1), jnp.float32),
  )(xta, xtb, wvt, W1, b1, W2, b2, W3, b3, Wf, bf)


def kernel(inputs, tables, w_lin, W1, b1, W2, b2, W3, b3, Wf, bf):
  idx_t = inputs.astype(jnp.int32).T
  t2a = tables[:NRA // D].transpose(0, 2, 1).reshape(NRA, V)
  t2b = tables[NRA // D:].transpose(0, 2, 1).reshape(NRB, V)
  w2 = w_lin.T
  xta = _sc_deep(t2a, idx_t, 0, NRA // NW)
  xtb = _sc_deep(t2b, idx_t, NRA, NRB // NW)
  wvt = _sc_wide(idx_t, w2)
  return _tc_mlp(xta, xtb, wvt, W1, b1.reshape(1, 256), W2, b2.reshape(1, 128),
                 W3, b3.reshape(1, 64), Wf, bf.reshape(1, 1))


# speedup vs baseline: 1.1793x; 1.1793x over previous
import functools

import jax
import jax.numpy as jnp
from jax import lax
from jax.experimental import pallas as pl
from jax.experimental.pallas import tpu as pltpu
import jax.experimental.pallas.tpu_sc as plsc

B = 16384
F = 26
V = 100000
D = 16

NC = 2
NS = 16
NW = NC * NS
L = 16

FD = F * D
RPW = FD // NW
WCH = 2048
NCH = B // WCH
NWIDE = F * NCH
KMAX = RPW + (NWIDE + NW - 1) // NW
SUB = 128


def _sc_wide(idx_t, w2):
  mesh = plsc.VectorSubcoreMesh(
      core_axis_name="c", subcore_axis_name="s", num_cores=NC, num_subcores=NS
  )

  @functools.partial(
      pl.kernel,
      out_type=jax.ShapeDtypeStruct((F, B), jnp.float32),
      mesh=mesh,
      compiler_params=pltpu.CompilerParams(use_tc_tiling_on_sc=False),
      scratch_types=[
          pltpu.VMEM((WCH,), jnp.int32),
          pltpu.VMEM((WCH,), jnp.float32),
          pltpu.SemaphoreType.DMA,
      ],
  )
  def k(idxt, w, wT, idx_v, out_v, sem):
    wid = lax.axis_index("s") * NC + lax.axis_index("c")

    def wide_task(wt):
      f = wt // NCH
      c = wt % NCH
      pltpu.sync_copy(idxt.at[f, pl.ds(c * WCH, WCH)], idx_v)

      def addoff(i, carry):
        idx_v[pl.ds(i * L, L)] = idx_v[pl.ds(i * L, L)] + f * V
        return carry

      lax.fori_loop(0, WCH // L, addoff, 0)

      def subgather(g, carry):
        base = g * (8 * SUB)
        cps = []
        for t in range(8):
          ii = idx_v.at[pl.ds(base + t * SUB, SUB)]
          cps.append(pltpu.async_copy(
              w.at[0].at[ii], out_v.at[pl.ds(base + t * SUB, SUB)], sem))
        for cp in cps:
          cp.wait()
        return carry

      lax.fori_loop(0, WCH // (8 * SUB), subgather, 0)
      pltpu.sync_copy(out_v, wT.at[f, pl.ds(c * WCH, WCH)])

    def round_body(kk, carry):
      wt = wid + NW * kk

      @pl.when(wt < NWIDE)
      def _():
        wide_task(wt)

      return carry

    lax.fori_loop(0, (NWIDE + NW - 1) // NW, round_body, 0)

  return k(idx_t, w2)


def _sc_deep(table2, idx_t, fd_base, rpw):
  nrows = rpw * NW
  mesh = plsc.VectorSubcoreMesh(
      core_axis_name="c", subcore_axis_name="s", num_cores=NC, num_subcores=NS
  )

  @functools.partial(
      pl.kernel,
      out_type=jax.ShapeDtypeStruct((nrows, B), jnp.float32),
      mesh=mesh,
      compiler_params=pltpu.CompilerParams(use_tc_tiling_on_sc=False),
      scratch_types=[
          pltpu.VMEM_SHARED((NS, V), jnp.float32),
          pltpu.VMEM((2, WCH), jnp.int32),
          pltpu.VMEM((2, WCH), jnp.float32),
          pltpu.SemaphoreType.DMA,
          pltpu.SemaphoreType.DMA,
          pltpu.SemaphoreType.DMA,
      ],
  )
  def k(t2, idxt, xT, tab_s, idx_v, out_v, sem_g, sem_i, sem_o):
    sid = lax.axis_index("s")
    wid = sid * NC + lax.axis_index("c")
    my_tab = tab_s.at[sid]

    def deep_row(r, carry):
      fd = wid * rpw + r
      f = (fd_base + fd) // D
      pltpu.sync_copy(t2.at[fd_base + fd], my_tab)
      pltpu.async_copy(idxt.at[f, pl.ds(0, WCH)], idx_v.at[0], sem_i).wait()

      def chunk(c, carry):
        cur = lax.rem(c, 2)
        nxt = lax.rem(c + 1, 2)

        @pl.when(c + 1 < NCH)
        def _():
          pltpu.async_copy(
              idxt.at[f, pl.ds((c + 1) * WCH, WCH)], idx_v.at[nxt], sem_i)

        def subgather(g, carry2):
          base = g * (8 * SUB)
          cps = []
          for t in range(8):
            ii = idx_v.at[cur].at[pl.ds(base + t * SUB, SUB)]
            cps.append(pltpu.async_copy(
                my_tab.at[ii], out_v.at[cur].at[pl.ds(base + t * SUB, SUB)],
                sem_g))
          for cp in cps:
            cp.wait()
          return carry2

        lax.fori_loop(0, WCH // (8 * SUB), subgather, 0)
        pltpu.sync_copy(out_v.at[cur], xT.at[fd, pl.ds(c * WCH, WCH)])

        @pl.when(c + 1 < NCH)
        def _():
          pltpu.make_async_copy(
              idxt.at[f, pl.ds((c + 1) * WCH, WCH)], idx_v.at[nxt], sem_i
          ).wait()

        return carry

      lax.fori_loop(0, NCH, chunk, 0)
      return carry

    lax.fori_loop(0, rpw, deep_row, 0)

  return k(table2, idx_t)


BM = 1024


NRA = 224
NRB = FD - NRA


def _mlp_body(xta_ref, xtb_ref, wvt_ref, W1_ref, b1_ref, W2_ref, b2_ref,
              W3_ref, b3_ref, Wf_ref, bf_ref, out_ref):
  cdims = (((0,), (0,)), ((), ()))
  W1 = W1_ref[...]
  pre = (lax.dot_general(xta_ref[...], W1[0:NRA, :], cdims,
                         preferred_element_type=jnp.float32)
         + lax.dot_general(xtb_ref[...], W1[NRA:FD, :], cdims,
                           preferred_element_type=jnp.float32))
  h = jnp.maximum(pre + b1_ref[...], 0.0)
  h = jnp.maximum(jnp.dot(h, W2_ref[...], preferred_element_type=jnp.float32)
                  + b2_ref[...], 0.0)
  h = jnp.maximum(jnp.dot(h, W3_ref[...], preferred_element_type=jnp.float32)
                  + b3_ref[...], 0.0)
  deep = jnp.dot(h, Wf_ref[...], preferred_element_type=jnp.float32) + bf_ref[...]
  ones = jnp.full((F, 1), 1.0, dtype=jnp.float32)
  wide = lax.dot_general(wvt_ref[...], ones, cdims,
                         preferred_element_type=jnp.float32)
  z = 0.5 * wide + 0.5 * deep
  out_ref[...] = 1.0 / (1.0 + jnp.exp(-z))


def _tc_mlp(xta, xtb, wvt, W1, b1, W2, b2, W3, b3, Wf, bf):
  grid = (B // BM,)
  return pl.pallas_call(
      _mlp_body,
      grid=grid,
      in_specs=[
          pl.BlockSpec((NRA, BM), lambda i: (0, i)),
          pl.BlockSpec((NRB, BM), lambda i: (0, i)),
          pl.BlockSpec((F, BM), lambda i: (0, i)),
          pl.BlockSpec((FD, 256), lambda i: (0, 0)),
          pl.BlockSpec((1, 256), lambda i: (0, 0)),
          pl.BlockSpec((256, 128), lambda i: (0, 0)),
          pl.BlockSpec((1, 128), lambda i: (0, 0)),
          pl.BlockSpec((128, 64), lambda i: (0, 0)),
          pl.BlockSpec((1, 64), lambda i: (0, 0)),
          pl.BlockSpec((64, 1), lambda i: (0, 0)),
          pl.BlockSpec((1, 1), lambda i: (0, 0)),
      ],
      out_specs=pl.BlockSpec((BM, 1), lambda i: (i, 0)),
      out_shape=jax.ShapeDtypeStruct((B, 1), jnp.float32),
  )(xta, xtb, wvt, W1, b1, W2, b2, W3, b3, Wf, bf)


def kernel(inputs, tables, w_lin, W1, b1, W2, b2, W3, b3, Wf, bf):
  idx_t = inputs.astype(jnp.int32).T
  table2 = tables.transpose(0, 2, 1).reshape(FD, V)
  w2 = w_lin.T
  xta = _sc_deep(table2, idx_t, 0, NRA // NW)
  xtb = _sc_deep(table2, idx_t, NRA, NRB // NW)
  wvt = _sc_wide(idx_t, w2)
  return _tc_mlp(xta, xtb, wvt, W1, b1.reshape(1, 256), W2, b2.reshape(1, 128),
                 W3, b3.reshape(1, 64), Wf, bf.reshape(1, 1))
